# Initial kernel scaffold; baseline (speedup 1.0000x reference)
import jax
import jax.numpy as jnp
from jax.experimental import pallas as pl


def kernel(logits, y, batch):
    raise NotImplementedError("write your pallas kernel here")


# speedup vs baseline: 5.1367x; 5.1367x over previous
import functools

import jax
import jax.numpy as jnp
from jax import lax
from jax.experimental import pallas as pl
from jax.experimental.pallas import tpu as pltpu
from jax.experimental.pallas import tpu_sc as plsc

N = 100000
J = 16
L = 16
NC = 2
NS = 16
W = NC * NS
GROUPS = N // L
GP_BASE = GROUPS // W
GP_EXTRA = GROUPS - GP_BASE * W
MAXG = GP_BASE + 1
MAXN = MAXG * L
G = 1024
ROW = 32
ACC = G * ROW
ACC_PER_SUB = ACC // NS


def _f_kernel(x_ref, o_ref):
    x = x_ref[...]
    o_ref[...] = jnp.maximum(x, 0.0) + jnp.log1p(jnp.exp(-jnp.abs(x)))


def _fin_kernel(acc_ref, b_ref, o_ref):
    a = acc_ref[0] + acc_ref[1]
    s = a[:, 0:16]
    mx = jnp.max(s, axis=1, keepdims=True)
    f_sum = a[:, 16:17]
    cnt = a[:, 17:18]
    rows = lax.broadcasted_iota(jnp.int32, (G, 1), 0)
    ng = jnp.max(b_ref[...]) + 1
    val = jnp.where((cnt > 0.0) & (rows < ng),
                    (f_sum - mx) / jnp.maximum(cnt, 1.0), 0.0)
    o_ref[...] = jnp.full((1, 1), jnp.sum(val) / ng.astype(jnp.float32))


def _bcast_lane(vec, j):
    idx = jnp.full((L,), j, jnp.int32)
    return vec.at[idx].get(mode="promise_in_bounds")


def _sc_body(x_hbm, f_hbm, b_hbm, y_hbm, out_hbm,
             x_v, f_v, g_v, y_v, stg_v, idx_v, zb_v, acc_sh, sem):
    cid = lax.axis_index("c")
    sid = lax.axis_index("s")
    wid = cid * NS + sid

    gs = GP_BASE * wid + jnp.minimum(wid, GP_EXTRA)
    ngroups = jnp.where(wid < GP_EXTRA, GP_BASE + 1, GP_BASE)
    off = jnp.minimum(gs * L, N - MAXN)
    lo = gs * L - off

    c1 = pltpu.async_copy(x_hbm.at[pl.ds(off, MAXN)], x_v, sem)
    c2 = pltpu.async_copy(f_hbm.at[pl.ds(off, MAXN)], f_v, sem)
    c3 = pltpu.async_copy(b_hbm.at[pl.ds(off, MAXN)], g_v, sem)
    c4 = pltpu.async_copy(y_hbm.at[pl.ds(off * J, MAXN * J)], y_v, sem)

    li = lax.iota(jnp.int32, L)
    zf = jnp.zeros((L,), jnp.float32)

    def zero_zb(r, _):
        zb_v[pl.ds(r * L, L)] = zf
        return 0

    lax.fori_loop(0, ACC_PER_SUB // L, zero_zb, 0)

    pltpu.sync_copy(zb_v, acc_sh.at[pl.ds(sid * ACC_PER_SUB, ACC_PER_SUB)])
    c1.wait()
    c2.wait()
    c3.wait()
    c4.wait()
    plsc.subcore_barrier()

    lane0 = li == 0
    lane1 = li == 1
    zeros = jnp.zeros((L,), jnp.float32)
    ones = jnp.ones((L,), jnp.float32)

    gv0 = g_v[pl.ds(lo, L)].astype(jnp.float32)
    prev_g0 = jnp.sum(jnp.where(lane0, gv0, jnp.zeros_like(gv0)))

    def do_flush(pg, acc_s, acc_fc):
        base = pg.astype(jnp.int32) * ROW
        stg_v[pl.ds(0, L)] = acc_s
        stg_v[pl.ds(L, L)] = acc_fc
        idx_v[pl.ds(0, L)] = base + li
        idx_v[pl.ds(L, L)] = base + L + li
        pltpu.sync_copy(stg_v, acc_sh.at[idx_v], add=True)

    def body(i, carry):
        prev_g, acc_s, acc_fc = carry
        nb = lo + i * L
        gv = g_v[pl.ds(nb, L)].astype(jnp.float32)
        xv = x_v[pl.ds(nb, L)]
        fv = f_v[pl.ds(nb, L)]
        gmn = jnp.min(gv)
        gmx = jnp.max(gv)

        def fast(args):
            prev_g, acc_s, acc_fc = args
            changed = gmn != prev_g

            @pl.when(changed)
            def _():
                do_flush(prev_g, acc_s, acc_fc)

            pv = jnp.full((L,), changed)
            acc_s = jnp.where(pv, zeros, acc_s)
            acc_fc = jnp.where(pv, zeros, acc_fc)
            for j in range(L):
                ycv = y_v[pl.ds((nb + j) * J, L)].astype(jnp.float32)
                acc_s = acc_s + _bcast_lane(xv, j) * ycv
            sf = jnp.sum(fv)
            acc_fc = acc_fc + jnp.where(lane0, jnp.full((L,), sf), zeros) \
                            + jnp.where(lane1, ones * float(L), zeros)
            return gmn, acc_s, acc_fc

        def slow(args):
            prev_g, acc_s, acc_fc = args
            for j in range(L):
                gb = _bcast_lane(gv, j)
                gj = jnp.sum(jnp.where(lane0, gb, zeros))
                changed = gj != prev_g

                @pl.when(changed)
                def _(pgx=prev_g, asx=acc_s, afx=acc_fc):
                    do_flush(pgx, asx, afx)

                pv = jnp.full((L,), changed)
                acc_s = jnp.where(pv, zeros, acc_s)
                acc_fc = jnp.where(pv, zeros, acc_fc)
                ycv = y_v[pl.ds((nb + j) * J, L)].astype(jnp.float32)
                acc_s = acc_s + _bcast_lane(xv, j) * ycv
                fb = _bcast_lane(fv, j)
                acc_fc = acc_fc + jnp.where(lane0, fb, zeros) \
                                + jnp.where(lane1, ones, zeros)
                prev_g = jnp.where(changed, gj, prev_g)
            return prev_g, acc_s, acc_fc

        return lax.cond(gmn == gmx, fast, slow,
                        (prev_g, acc_s, acc_fc))

    prev_g, acc_s, acc_fc = lax.fori_loop(
        0, ngroups, body, (prev_g0, zeros, zeros))
    do_flush(prev_g, acc_s, acc_fc)
    plsc.subcore_barrier()

    pltpu.sync_copy(acc_sh.at[pl.ds(sid * ACC_PER_SUB, ACC_PER_SUB)],
                    out_hbm.at[cid, pl.ds(sid * ACC_PER_SUB, ACC_PER_SUB)])


@functools.partial(
    pl.kernel,
    out_type=jax.ShapeDtypeStruct((NC, ACC), jnp.float32),
    mesh=plsc.VectorSubcoreMesh(core_axis_name="c", subcore_axis_name="s"),
    compiler_params=pltpu.CompilerParams(needs_layout_passes=False),
    scratch_types=[
        pltpu.VMEM((MAXN,), jnp.float32),
        pltpu.VMEM((MAXN,), jnp.float32),
        pltpu.VMEM((MAXN,), jnp.int32),
        pltpu.VMEM((MAXN * J,), jnp.int32),
        pltpu.VMEM((2 * L,), jnp.float32),
        pltpu.VMEM((2 * L,), jnp.int32),
        pltpu.VMEM((ACC_PER_SUB,), jnp.float32),
        pltpu.VMEM_SHARED((ACC,), jnp.float32),
        pltpu.SemaphoreType.DMA,
    ],
)
def _sc_call(x_hbm, f_hbm, b_hbm, y_hbm, out_hbm,
             x_v, f_v, g_v, y_v, stg_v, idx_v, zb_v, acc_sh, sem):
    _sc_body(x_hbm, f_hbm, b_hbm, y_hbm, out_hbm,
             x_v, f_v, g_v, y_v, stg_v, idx_v, zb_v, acc_sh, sem)


def kernel(logits, y, batch):
    x = logits.reshape(N).astype(jnp.float32)
    yi = y.astype(jnp.int32).reshape(N * J)
    bi = batch.astype(jnp.int32)

    x2d = x.reshape(800, 125)
    f2d = pl.pallas_call(
        _f_kernel,
        out_shape=jax.ShapeDtypeStruct((800, 125), jnp.float32),
    )(x2d)
    f = f2d.reshape(N)

    acc = _sc_call(x, f, bi, yi).reshape(NC, G, ROW)

    res = pl.pallas_call(
        _fin_kernel,
        out_shape=jax.ShapeDtypeStruct((1, 1), jnp.float32),
    )(acc, bi.reshape(800, 125))
    return res[0, 0]
